# Initial kernel scaffold; baseline (speedup 1.0000x reference)
import jax
import jax.numpy as jnp
from jax.experimental import pallas as pl


def kernel(xs, ys, maps):
    raise NotImplementedError("write your pallas kernel here")


# speedup vs baseline: 6.1690x; 6.1690x over previous
import functools

import jax
import jax.numpy as jnp
from jax import lax
from jax.experimental import pallas as pl
from jax.experimental.pallas import tpu as pltpu
from jax.experimental.pallas import tpu_sc as plsc

NC, NS = 2, 16
NW = NC * NS
D = 16 * 16 * 16
TOK = 2 * 2048
TPW = TOK // NW
CHUNK = 8
NCHUNK = TPW // CHUNK


def _body(table, xi, yi, outx, outy,
          idx_x, idx_y, buf0, buf1, gsem0, gsem1, wsem0, wsem1):
    wid = lax.axis_index("s") * NC + lax.axis_index("c")
    base = wid * TPW

    pltpu.sync_copy(xi.at[wid], idx_x)
    pltpu.sync_copy(yi.at[wid], idx_y)

    bufs = (buf0, buf1)
    gsems = (gsem0, gsem1)
    wsems = (wsem0, wsem1)

    for idx_v, out in ((idx_x, outx), (idx_y, outy)):
        writes = [None, None]
        gathers = [None, None]
        gathers[0] = pltpu.async_copy(table.at[idx_v.at[0]], bufs[0], gsems[0])
        for j in range(NCHUNK):
            b = j % 2
            gathers[b].wait()
            if j + 1 < NCHUNK:
                nb = 1 - b
                if writes[nb] is not None:
                    writes[nb].wait()
                gathers[nb] = pltpu.async_copy(
                    table.at[idx_v.at[j + 1]], bufs[nb], gsems[nb])
            writes[b] = pltpu.async_copy(
                bufs[b], out.at[pl.ds(base + j * CHUNK, CHUNK)], wsems[b])
        writes[0].wait()
        writes[1].wait()


@jax.jit
def _gather_sc(table, xi, yi):
    f = pl.kernel(
        _body,
        out_type=(
            jax.ShapeDtypeStruct((TOK, D), jnp.float32),
            jax.ShapeDtypeStruct((TOK, D), jnp.float32),
        ),
        mesh=plsc.VectorSubcoreMesh(core_axis_name="c", subcore_axis_name="s"),
        scratch_types=[
            pltpu.VMEM((NCHUNK, CHUNK), jnp.int32),
            pltpu.VMEM((NCHUNK, CHUNK), jnp.int32),
            pltpu.VMEM((CHUNK, D), jnp.float32),
            pltpu.VMEM((CHUNK, D), jnp.float32),
            pltpu.SemaphoreType.DMA,
            pltpu.SemaphoreType.DMA,
            pltpu.SemaphoreType.DMA,
            pltpu.SemaphoreType.DMA,
        ],
    )
    return f(table, xi, yi)


def kernel(xs, ys, maps):
    size1, na, nh, dim, _ = maps.shape
    b, s = xs.shape
    table = maps.reshape(size1 * na, nh * dim * dim)
    xi = (xs.reshape(-1) * 2).reshape(NW, NCHUNK, CHUNK).astype(jnp.int32)
    yi = (ys.reshape(-1) * 2 + 1).reshape(NW, NCHUNK, CHUNK).astype(jnp.int32)
    out_x, out_y = _gather_sc(table, xi, yi)
    return (out_x.reshape(b, s, nh, dim, dim), out_y.reshape(b, s, nh, dim, dim))


# speedup vs baseline: 16.9051x; 2.7404x over previous
import functools

import jax
import jax.numpy as jnp
from jax import lax
from jax.experimental import pallas as pl
from jax.experimental.pallas import tpu as pltpu
from jax.experimental.pallas import tpu_sc as plsc

NC, NS = 2, 16
NW = NC * NS
H, R, C = 16, 16, 16
SEQ = 2048
NTOK = 2 * SEQ
PAIRS_PER_W = (H * R) // NW
LANES = 16


def _body(mt, xi, yi, ox, oy,
          idx_x, idx_y, in0, in1, ob0, ob1, gs0, gs1, ws0, ws1):
    wid = lax.axis_index("s") * NC + lax.axis_index("c")

    pltpu.sync_copy(xi, idx_x)
    pltpu.sync_copy(yi, idx_y)

    ins = (in0, in1)
    gsems = (gs0, gs1)
    obufs = (ob0, ob1)
    wsems = (ws0, ws1)

    pieces = [(k, a, cg) for k in range(PAIRS_PER_W)
              for a in range(2) for cg in range(2)]

    def src_slice(k, a, cg):
        p = wid * PAIRS_PER_W + k
        h = p // R
        r = p % R
        return mt.at[a, h, r, pl.ds(cg * 8, 8), :], h, r

    def start_gather(pi, ib):
        k, a, cg = pieces[pi]
        src, _, _ = src_slice(k, a, cg)
        return pltpu.async_copy(src, ins[ib], gsems[ib])

    def compute(pi, ib, b, ob):
        _, a, _ = pieces[pi]
        idx_v = idx_x if a == 0 else idx_y
        in_b = ins[ib]
        out_b = obufs[ob]

        def loop(sg, _):
            ivec = idx_v[pl.ds(b * SEQ + sg * LANES, LANES)]
            for c in range(8):
                cvec = jnp.full((LANES,), c, jnp.int32)
                vals = plsc.load_gather(in_b, [cvec, ivec])
                out_b[c, pl.ds(sg * LANES, LANES)] = vals
            return _

        lax.fori_loop(0, SEQ // LANES, loop, None)

    def start_write(pi, b, ob):
        k, a, cg = pieces[pi]
        p = wid * PAIRS_PER_W + k
        h = p // R
        r = p % R
        dst_ref = ox if a == 0 else oy
        dst = dst_ref.at[b, h, r, pl.ds(cg * 8, 8), :]
        return pltpu.async_copy(obufs[ob], dst, wsems[ob])

    writes = [None, None]
    g = [None, None]
    g[0] = start_gather(0, 0)
    for pi in range(len(pieces)):
        ib = pi % 2
        g[ib].wait()
        if pi + 1 < len(pieces):
            g[1 - ib] = start_gather(pi + 1, 1 - ib)
        for b in range(2):
            ob = b
            if writes[ob] is not None:
                writes[ob].wait()
            compute(pi, ib, b, ob)
            writes[ob] = start_write(pi, b, ob)
    writes[0].wait()
    writes[1].wait()


@jax.jit
def _gather_sc(mt, xi, yi):
    f = pl.kernel(
        _body,
        out_type=(
            jax.ShapeDtypeStruct((2, H, R, C, SEQ), jnp.float32),
            jax.ShapeDtypeStruct((2, H, R, C, SEQ), jnp.float32),
        ),
        mesh=plsc.VectorSubcoreMesh(core_axis_name="c", subcore_axis_name="s"),
        scratch_types=[
            pltpu.VMEM((NTOK,), jnp.int32),
            pltpu.VMEM((NTOK,), jnp.int32),
            pltpu.VMEM((8, 2049), jnp.float32),
            pltpu.VMEM((8, 2049), jnp.float32),
            pltpu.VMEM((8, SEQ), jnp.float32),
            pltpu.VMEM((8, SEQ), jnp.float32),
            pltpu.SemaphoreType.DMA,
            pltpu.SemaphoreType.DMA,
            pltpu.SemaphoreType.DMA,
            pltpu.SemaphoreType.DMA,
        ],
        compiler_params=pltpu.CompilerParams(
            use_tc_tiling_on_sc=True, needs_layout_passes=False),
    )
    return f(mt, xi, yi)


def kernel(xs, ys, maps):
    mt = jnp.transpose(maps, (1, 2, 3, 4, 0))
    xi = xs.reshape(-1).astype(jnp.int32)
    yi = ys.reshape(-1).astype(jnp.int32)
    ox_t, oy_t = _gather_sc(mt, xi, yi)
    out_x = jnp.transpose(ox_t, (0, 4, 1, 2, 3))
    out_y = jnp.transpose(oy_t, (0, 4, 1, 2, 3))
    return (out_x, out_y)
